# baseline (speedup 1.0000x reference)
import functools

import jax
import jax.numpy as jnp
from jax import lax
from jax.experimental import pallas as pl
from jax.experimental.pallas import tpu as pltpu
from jax.experimental.pallas import tpu_sc as plsc

N = 10000
E = 320000
DM = 128
SMOOTH = 0.5

NC = 2
NS = 16
NW = NC * NS
BE = 64
NB = 160
WIN = 16
NWIN = NB // WIN
WIN_C0 = 7
WIN_C1 = 13
EPW = NB * BE
EP = NW * EPW
NPAD = 10240
STRIPE = NPAD // NS

_mesh = plsc.VectorSubcoreMesh(core_axis_name="c", subcore_axis_name="s")


@functools.partial(
    pl.kernel,
    mesh=_mesh,
    out_type=jax.ShapeDtypeStruct((NC, NPAD), jnp.float32),
    scratch_types=[
        pltpu.VMEM((NB, BE), jnp.int32),
        pltpu.VMEM((NB, BE), jnp.float32),
        pltpu.VMEM_SHARED((NPAD,), jnp.float32),
    ],
)
def _deg_kernel(col_h, val_h, zero1_h, deg_out, col_t, val_t, sh_deg):
    c = lax.axis_index("c")
    s = lax.axis_index("s")
    w = c * NS + s
    pltpu.sync_copy(col_h.at[w], col_t)
    pltpu.sync_copy(val_h.at[w], val_t)
    pltpu.sync_copy(zero1_h.at[pl.ds(s * STRIPE, STRIPE)],
                    sh_deg.at[pl.ds(s * STRIPE, STRIPE)])
    plsc.subcore_barrier()

    def _scat(b, carry):
        pltpu.sync_copy(val_t.at[b], sh_deg.at[col_t.at[b]], add=True)
        return carry

    lax.fori_loop(0, NB, _scat, 0)
    plsc.subcore_barrier()
    pltpu.sync_copy(sh_deg.at[pl.ds(s * STRIPE, STRIPE)],
                    deg_out.at[c, pl.ds(s * STRIPE, STRIPE)])


@functools.partial(
    pl.kernel,
    mesh=_mesh,
    out_type=jax.ShapeDtypeStruct((NC * NPAD, DM), jnp.float32),
    compiler_params=pltpu.CompilerParams(
        needs_layout_passes=False, use_tc_tiling_on_sc=False),
    scratch_types=[
        pltpu.VMEM((2, WIN, BE), jnp.int32),
        pltpu.VMEM((2, WIN, BE), jnp.int32),
        pltpu.VMEM((2, WIN, BE), jnp.float32),
        pltpu.VMEM((4, BE, DM // 2), jnp.int32),
        pltpu.VMEM((2, BE, DM), jnp.float32),
        pltpu.VMEM_SHARED((NPAD, DM), jnp.float32),
        [pltpu.SemaphoreType.DMA] * 4,
        [pltpu.SemaphoreType.DMA] * 2,
        pltpu.SemaphoreType.DMA,
    ],
)
def _spmm_kernel(hp_h, row_h, col_h, val_h, zero2_h, agg_out,
                 row_t, col_t, val_t, rows_t, sc_t, sh_agg, sg, ss, si):
    c = lax.axis_index("c")
    s = lax.axis_index("s")
    nwin_local = jnp.where(c == 0, WIN_C0, WIN_C1)
    chunk0 = c * (NS * WIN_C0) + s * nwin_local
    base = s * STRIPE
    for k in range(STRIPE // 128):
        pltpu.sync_copy(zero2_h, sh_agg.at[pl.ds(base + k * 128, 128)])
    plsc.subcore_barrier()

    pltpu.sync_copy(row_h.at[chunk0], row_t.at[0])
    pltpu.sync_copy(col_h.at[chunk0], col_t.at[0])
    pltpu.sync_copy(val_h.at[chunk0], val_t.at[0])
    pltpu.async_copy(row_h.at[chunk0 + 1], row_t.at[1], si)
    pltpu.async_copy(col_h.at[chunk0 + 1], col_t.at[1], si)
    pltpu.async_copy(val_h.at[chunk0 + 1], val_t.at[1], si)
    for kb in range(4):
        pltpu.async_copy(hp_h.at[col_t.at[0, kb]], rows_t.at[kb], sg[kb])

    def _drain_si(wbn):
        pltpu.make_async_copy(row_h.at[0], row_t.at[wbn], si).wait()
        pltpu.make_async_copy(col_h.at[0], col_t.at[wbn], si).wait()
        pltpu.make_async_copy(val_h.at[0], val_t.at[wbn], si).wait()

    def _win(win, carry):
        wb = win % 2
        wbn = 1 - wb

        def _grp(i, carry2):
            for kb in range(4):
                b = 4 * i + kb
                bg = win * WIN + b
                kc = kb % 2
                pltpu.make_async_copy(
                    hp_h.at[col_t.at[wb, b]], rows_t.at[kb], sg[kb]).wait()
                @pl.when(bg >= 2)
                def _():
                    pltpu.make_async_copy(
                        sc_t.at[kc], sh_agg.at[row_t.at[wb, b]], ss[kc]).wait()
                for j2 in range(BE // 16):
                    cv = val_t[wb, b, pl.ds(j2 * 16, 16)]
                    for j in range(16):
                        e = j2 * 16 + j
                        cf = cv[j]
                        for k in range(DM // 32):
                            u = rows_t[kb, e, pl.ds(k * 16, 16)]
                            lo = lax.bitcast_convert_type(
                                u << 16, jnp.float32)
                            hi = lax.bitcast_convert_type(
                                u & jnp.int32(-65536), jnp.float32)
                            sc_t[kc, e, pl.ds(k * 32, 16)] = lo * cf
                            sc_t[kc, e, pl.ds(k * 32 + 16, 16)] = hi * cf
                pltpu.async_copy(
                    sc_t.at[kc], sh_agg.at[row_t.at[wb, b]], ss[kc],
                    add=True)
                if kb == 3:
                    @pl.when((i == 0) & (win >= 1) & (win + 1 < nwin_local))
                    def _():
                        chn = chunk0 + win + 1
                        pltpu.async_copy(row_h.at[chn], row_t.at[wbn], si)
                        pltpu.async_copy(col_h.at[chn], col_t.at[wbn], si)
                        pltpu.async_copy(val_h.at[chn], val_t.at[wbn], si)
                if kb == 0:
                    @pl.when((i == 3) & (win + 1 < nwin_local))
                    def _():
                        _drain_si(wbn)
                @pl.when(i < 3)
                def _():
                    pltpu.async_copy(
                        hp_h.at[col_t.at[wb, b + 4]], rows_t.at[kb], sg[kb])
                @pl.when((i == 3) & (win + 1 < nwin_local))
                def _():
                    pltpu.async_copy(
                        hp_h.at[col_t.at[wbn, kb]], rows_t.at[kb], sg[kb])
            return carry2

        lax.fori_loop(0, WIN // 4, _grp, 0)
        return carry

    lax.fori_loop(0, nwin_local, _win, 0)
    for kc in (0, 1):
        pltpu.make_async_copy(
            sc_t.at[kc], sh_agg.at[row_t.at[0, WIN - 2 + kc]], ss[kc]).wait()
    plsc.subcore_barrier()
    for k in range(STRIPE // 128):
        off = base + k * 128
        pltpu.sync_copy(sh_agg.at[pl.ds(off, 128)],
                        agg_out.at[pl.ds(c * NPAD + off, 128)])


_RB = 1000


def _support_body(x_ref, wt_ref, b_ref, d0_ref, d1_ref, sup_ref, hp_ref, dc_ref):
    sup = jnp.dot(x_ref[...], wt_ref[...], preferred_element_type=jnp.float32)
    sup = sup + b_ref[...]
    d = lax.rsqrt(d0_ref[...] + d1_ref[...] + 1.0)
    sup_ref[...] = sup
    hp_ref[...] = sup * d
    dc_ref[...] = d


def _combine_body(a0_ref, a1_ref, hp_ref, sup_ref, dc_ref, out_ref):
    acc = (a0_ref[...] + a1_ref[...] + hp_ref[...]) * dc_ref[...]
    out_ref[...] = acc * (SMOOTH / (1.0 + SMOOTH)) + sup_ref[...] * (1.0 / (1.0 + SMOOTH))


def kernel(x, adj_indices, adj_values, W, b):
    row = adj_indices[0]
    col = adj_indices[1]
    pad = EP - E
    row_p = jnp.pad(row, (0, pad)).reshape(NW * NWIN, WIN, BE)
    col_p = jnp.pad(col, (0, pad)).reshape(NW * NWIN, WIN, BE)
    val_p = jnp.pad(adj_values, (0, pad)).reshape(NW * NWIN, WIN, BE)
    col_p4 = col_p.reshape(NW, NB, BE)
    val_p4 = val_p.reshape(NW, NB, BE)
    zero1 = jnp.zeros((NPAD,), jnp.float32)
    zero2 = jnp.zeros((128, DM), jnp.float32)

    deg_p = _deg_kernel(col_p4, val_p4, zero1)
    d0 = deg_p[0, :N, None]
    d1 = deg_p[1, :N, None]

    wt = W.T
    b2 = b.reshape(1, DM)
    grid = (N // _RB,)
    sup, hp, dc = pl.pallas_call(
        _support_body,
        grid=grid,
        in_specs=[
            pl.BlockSpec((_RB, DM), lambda i: (i, 0)),
            pl.BlockSpec((DM, DM), lambda i: (0, 0)),
            pl.BlockSpec((1, DM), lambda i: (0, 0)),
            pl.BlockSpec((_RB, 1), lambda i: (i, 0)),
            pl.BlockSpec((_RB, 1), lambda i: (i, 0)),
        ],
        out_specs=[
            pl.BlockSpec((_RB, DM), lambda i: (i, 0)),
            pl.BlockSpec((_RB, DM), lambda i: (i, 0)),
            pl.BlockSpec((_RB, 1), lambda i: (i, 0)),
        ],
        out_shape=[
            jax.ShapeDtypeStruct((N, DM), jnp.float32),
            jax.ShapeDtypeStruct((N, DM), jnp.float32),
            jax.ShapeDtypeStruct((N, 1), jnp.float32),
        ],
    )(x, wt, b2, d0, d1)

    hp_b = (hp.reshape(N, DM // 32, 2, 16).swapaxes(2, 3)
            .reshape(N, DM // 2, 2).astype(jnp.bfloat16))
    hp_i = jax.lax.bitcast_convert_type(hp_b, jnp.int32)
    agg = _spmm_kernel(hp_i, row_p, col_p, val_p, zero2)
    a0 = agg[:N]
    a1 = agg[NPAD:NPAD + N]

    out = pl.pallas_call(
        _combine_body,
        grid=grid,
        in_specs=[
            pl.BlockSpec((_RB, DM), lambda i: (i, 0)),
            pl.BlockSpec((_RB, DM), lambda i: (i, 0)),
            pl.BlockSpec((_RB, DM), lambda i: (i, 0)),
            pl.BlockSpec((_RB, DM), lambda i: (i, 0)),
            pl.BlockSpec((_RB, 1), lambda i: (i, 0)),
        ],
        out_specs=pl.BlockSpec((_RB, DM), lambda i: (i, 0)),
        out_shape=jax.ShapeDtypeStruct((N, DM), jnp.float32),
    )(a0, a1, hp, sup, dc)
    return out


# speedup vs baseline: 1.0956x; 1.0956x over previous
import functools

import jax
import jax.numpy as jnp
from jax import lax
from jax.experimental import pallas as pl
from jax.experimental.pallas import tpu as pltpu
from jax.experimental.pallas import tpu_sc as plsc

N = 10000
E = 320000
DM = 128
SMOOTH = 0.5

NC = 2
NS = 16
NW = NC * NS
BE = 64
NB = 160
WIN = 16
NWIN = NB // WIN
WIN_C0 = 13
WIN_C1 = 7
EPW = NB * BE
EP = NW * EPW
NPAD = 10240
STRIPE = NPAD // NS

_mesh = plsc.VectorSubcoreMesh(core_axis_name="c", subcore_axis_name="s")


@functools.partial(
    pl.kernel,
    mesh=_mesh,
    out_type=jax.ShapeDtypeStruct((NC, NPAD), jnp.float32),
    scratch_types=[
        pltpu.VMEM((NB, BE), jnp.int32),
        pltpu.VMEM((NB, BE), jnp.float32),
        pltpu.VMEM_SHARED((NPAD,), jnp.float32),
    ],
)
def _deg_kernel(col_h, val_h, zero1_h, deg_out, col_t, val_t, sh_deg):
    c = lax.axis_index("c")
    s = lax.axis_index("s")
    w = c * NS + s
    pltpu.sync_copy(col_h.at[w], col_t)
    pltpu.sync_copy(val_h.at[w], val_t)
    pltpu.sync_copy(zero1_h.at[pl.ds(s * STRIPE, STRIPE)],
                    sh_deg.at[pl.ds(s * STRIPE, STRIPE)])
    plsc.subcore_barrier()

    def _scat(b, carry):
        pltpu.sync_copy(val_t.at[b], sh_deg.at[col_t.at[b]], add=True)
        return carry

    lax.fori_loop(0, NB, _scat, 0)
    plsc.subcore_barrier()
    pltpu.sync_copy(sh_deg.at[pl.ds(s * STRIPE, STRIPE)],
                    deg_out.at[c, pl.ds(s * STRIPE, STRIPE)])


@functools.partial(
    pl.kernel,
    mesh=_mesh,
    out_type=jax.ShapeDtypeStruct((NC * NPAD, DM), jnp.float32),
    compiler_params=pltpu.CompilerParams(
        needs_layout_passes=False, use_tc_tiling_on_sc=False),
    scratch_types=[
        pltpu.VMEM((2, WIN, BE), jnp.int32),
        pltpu.VMEM((2, WIN, BE), jnp.int32),
        pltpu.VMEM((2, WIN, BE), jnp.float32),
        pltpu.VMEM((4, BE, DM // 2), jnp.int32),
        pltpu.VMEM((2, BE, DM), jnp.float32),
        pltpu.VMEM_SHARED((NPAD, DM), jnp.float32),
        [pltpu.SemaphoreType.DMA] * 4,
        [pltpu.SemaphoreType.DMA] * 2,
        pltpu.SemaphoreType.DMA,
    ],
)
def _spmm_kernel(hp_h, row_h, col_h, val_h, zero2_h, agg_out,
                 row_t, col_t, val_t, rows_t, sc_t, sh_agg, sg, ss, si):
    c = lax.axis_index("c")
    s = lax.axis_index("s")
    nwin_local = jnp.where(c == 0, WIN_C0, WIN_C1)
    chunk0 = c * (NS * WIN_C0) + s * nwin_local
    base = s * STRIPE
    for k in range(STRIPE // 128):
        pltpu.sync_copy(zero2_h, sh_agg.at[pl.ds(base + k * 128, 128)])
    plsc.subcore_barrier()

    pltpu.sync_copy(row_h.at[chunk0], row_t.at[0])
    pltpu.sync_copy(col_h.at[chunk0], col_t.at[0])
    pltpu.sync_copy(val_h.at[chunk0], val_t.at[0])
    pltpu.async_copy(row_h.at[chunk0 + 1], row_t.at[1], si)
    pltpu.async_copy(col_h.at[chunk0 + 1], col_t.at[1], si)
    pltpu.async_copy(val_h.at[chunk0 + 1], val_t.at[1], si)
    for kb in range(4):
        pltpu.async_copy(hp_h.at[col_t.at[0, kb]], rows_t.at[kb], sg[kb])

    def _drain_si(wbn):
        pltpu.make_async_copy(row_h.at[0], row_t.at[wbn], si).wait()
        pltpu.make_async_copy(col_h.at[0], col_t.at[wbn], si).wait()
        pltpu.make_async_copy(val_h.at[0], val_t.at[wbn], si).wait()

    def _win(win, carry):
        wb = win % 2
        wbn = 1 - wb

        def _grp(i, carry2):
            for kb in range(4):
                b = 4 * i + kb
                bg = win * WIN + b
                kc = kb % 2
                pltpu.make_async_copy(
                    hp_h.at[col_t.at[wb, b]], rows_t.at[kb], sg[kb]).wait()
                @pl.when(bg >= 2)
                def _():
                    pltpu.make_async_copy(
                        sc_t.at[kc], sh_agg.at[row_t.at[wb, b]], ss[kc]).wait()
                for j2 in range(BE // 16):
                    cv = val_t[wb, b, pl.ds(j2 * 16, 16)]
                    for j in range(16):
                        e = j2 * 16 + j
                        cf = cv[j]
                        for k in range(DM // 32):
                            u = rows_t[kb, e, pl.ds(k * 16, 16)]
                            lo = lax.bitcast_convert_type(
                                u << 16, jnp.float32)
                            hi = lax.bitcast_convert_type(
                                u & jnp.int32(-65536), jnp.float32)
                            sc_t[kc, e, pl.ds(k * 32, 16)] = lo * cf
                            sc_t[kc, e, pl.ds(k * 32 + 16, 16)] = hi * cf
                pltpu.async_copy(
                    sc_t.at[kc], sh_agg.at[row_t.at[wb, b]], ss[kc],
                    add=True)
                if kb == 3:
                    @pl.when((i == 0) & (win >= 1) & (win + 1 < nwin_local))
                    def _():
                        chn = chunk0 + win + 1
                        pltpu.async_copy(row_h.at[chn], row_t.at[wbn], si)
                        pltpu.async_copy(col_h.at[chn], col_t.at[wbn], si)
                        pltpu.async_copy(val_h.at[chn], val_t.at[wbn], si)
                if kb == 0:
                    @pl.when((i == 3) & (win + 1 < nwin_local))
                    def _():
                        _drain_si(wbn)
                @pl.when(i < 3)
                def _():
                    pltpu.async_copy(
                        hp_h.at[col_t.at[wb, b + 4]], rows_t.at[kb], sg[kb])
                @pl.when((i == 3) & (win + 1 < nwin_local))
                def _():
                    pltpu.async_copy(
                        hp_h.at[col_t.at[wbn, kb]], rows_t.at[kb], sg[kb])
            return carry2

        lax.fori_loop(0, WIN // 4, _grp, 0)
        return carry

    lax.fori_loop(0, nwin_local, _win, 0)
    for kc in (0, 1):
        pltpu.make_async_copy(
            sc_t.at[kc], sh_agg.at[row_t.at[0, WIN - 2 + kc]], ss[kc]).wait()
    plsc.subcore_barrier()
    for k in range(STRIPE // 128):
        off = base + k * 128
        pltpu.sync_copy(sh_agg.at[pl.ds(off, 128)],
                        agg_out.at[pl.ds(c * NPAD + off, 128)])


_RB = 1000


def _support_body(x_ref, wt_ref, b_ref, d0_ref, d1_ref, sup_ref, hp_ref, dc_ref):
    sup = jnp.dot(x_ref[...], wt_ref[...], preferred_element_type=jnp.float32)
    sup = sup + b_ref[...]
    d = lax.rsqrt(d0_ref[...] + d1_ref[...] + 1.0)
    sup_ref[...] = sup
    hp_ref[...] = sup * d
    dc_ref[...] = d


def _combine_body(a0_ref, a1_ref, hp_ref, sup_ref, dc_ref, out_ref):
    acc = (a0_ref[...] + a1_ref[...] + hp_ref[...]) * dc_ref[...]
    out_ref[...] = acc * (SMOOTH / (1.0 + SMOOTH)) + sup_ref[...] * (1.0 / (1.0 + SMOOTH))


def kernel(x, adj_indices, adj_values, W, b):
    row = adj_indices[0]
    col = adj_indices[1]
    pad = EP - E
    row_p = jnp.pad(row, (0, pad)).reshape(NW * NWIN, WIN, BE)
    col_p = jnp.pad(col, (0, pad)).reshape(NW * NWIN, WIN, BE)
    val_p = jnp.pad(adj_values, (0, pad)).reshape(NW * NWIN, WIN, BE)
    col_p4 = col_p.reshape(NW, NB, BE)
    val_p4 = val_p.reshape(NW, NB, BE)
    zero1 = jnp.zeros((NPAD,), jnp.float32)
    zero2 = jnp.zeros((128, DM), jnp.float32)

    deg_p = _deg_kernel(col_p4, val_p4, zero1)
    d0 = deg_p[0, :N, None]
    d1 = deg_p[1, :N, None]

    wt = W.T
    b2 = b.reshape(1, DM)
    grid = (N // _RB,)
    sup, hp, dc = pl.pallas_call(
        _support_body,
        grid=grid,
        in_specs=[
            pl.BlockSpec((_RB, DM), lambda i: (i, 0)),
            pl.BlockSpec((DM, DM), lambda i: (0, 0)),
            pl.BlockSpec((1, DM), lambda i: (0, 0)),
            pl.BlockSpec((_RB, 1), lambda i: (i, 0)),
            pl.BlockSpec((_RB, 1), lambda i: (i, 0)),
        ],
        out_specs=[
            pl.BlockSpec((_RB, DM), lambda i: (i, 0)),
            pl.BlockSpec((_RB, DM), lambda i: (i, 0)),
            pl.BlockSpec((_RB, 1), lambda i: (i, 0)),
        ],
        out_shape=[
            jax.ShapeDtypeStruct((N, DM), jnp.float32),
            jax.ShapeDtypeStruct((N, DM), jnp.float32),
            jax.ShapeDtypeStruct((N, 1), jnp.float32),
        ],
    )(x, wt, b2, d0, d1)

    hp_b = (hp.reshape(N, DM // 32, 2, 16).swapaxes(2, 3)
            .reshape(N, DM // 2, 2).astype(jnp.bfloat16))
    hp_i = jax.lax.bitcast_convert_type(hp_b, jnp.int32)
    agg = _spmm_kernel(hp_i, row_p, col_p, val_p, zero2)
    a0 = agg[:N]
    a1 = agg[NPAD:NPAD + N]

    out = pl.pallas_call(
        _combine_body,
        grid=grid,
        in_specs=[
            pl.BlockSpec((_RB, DM), lambda i: (i, 0)),
            pl.BlockSpec((_RB, DM), lambda i: (i, 0)),
            pl.BlockSpec((_RB, DM), lambda i: (i, 0)),
            pl.BlockSpec((_RB, DM), lambda i: (i, 0)),
            pl.BlockSpec((_RB, 1), lambda i: (i, 0)),
        ],
        out_specs=pl.BlockSpec((_RB, DM), lambda i: (i, 0)),
        out_shape=jax.ShapeDtypeStruct((N, DM), jnp.float32),
    )(a0, a1, hp, sup, dc)
    return out


# speedup vs baseline: 1.1309x; 1.0322x over previous
import functools

import jax
import jax.numpy as jnp
from jax import lax
from jax.experimental import pallas as pl
from jax.experimental.pallas import tpu as pltpu
from jax.experimental.pallas import tpu_sc as plsc

N = 10000
E = 320000
DM = 128
SMOOTH = 0.5

NC = 2
NS = 16
NW = NC * NS
BE = 64
NB = 160
WIN = 16
NWIN = NB // WIN
WIN_C0 = 12
WIN_C1 = 8
EPW = NB * BE
EP = NW * EPW
NPAD = 10240
STRIPE = NPAD // NS

_mesh = plsc.VectorSubcoreMesh(core_axis_name="c", subcore_axis_name="s")


@functools.partial(
    pl.kernel,
    mesh=_mesh,
    out_type=jax.ShapeDtypeStruct((NC, NPAD), jnp.float32),
    scratch_types=[
        pltpu.VMEM((NB, BE), jnp.int32),
        pltpu.VMEM((NB, BE), jnp.float32),
        pltpu.VMEM_SHARED((NPAD,), jnp.float32),
    ],
)
def _deg_kernel(col_h, val_h, zero1_h, deg_out, col_t, val_t, sh_deg):
    c = lax.axis_index("c")
    s = lax.axis_index("s")
    w = c * NS + s
    pltpu.sync_copy(col_h.at[w], col_t)
    pltpu.sync_copy(val_h.at[w], val_t)
    pltpu.sync_copy(zero1_h.at[pl.ds(s * STRIPE, STRIPE)],
                    sh_deg.at[pl.ds(s * STRIPE, STRIPE)])
    plsc.subcore_barrier()

    def _scat(b, carry):
        pltpu.sync_copy(val_t.at[b], sh_deg.at[col_t.at[b]], add=True)
        return carry

    lax.fori_loop(0, NB, _scat, 0)
    plsc.subcore_barrier()
    pltpu.sync_copy(sh_deg.at[pl.ds(s * STRIPE, STRIPE)],
                    deg_out.at[c, pl.ds(s * STRIPE, STRIPE)])


@functools.partial(
    pl.kernel,
    mesh=_mesh,
    out_type=jax.ShapeDtypeStruct((NC * NPAD, DM), jnp.float32),
    compiler_params=pltpu.CompilerParams(
        needs_layout_passes=False, use_tc_tiling_on_sc=False),
    scratch_types=[
        pltpu.VMEM((2, WIN, BE), jnp.int32),
        pltpu.VMEM((2, WIN, BE), jnp.int32),
        pltpu.VMEM((2, WIN, BE), jnp.float32),
        pltpu.VMEM((4, BE, DM // 2), jnp.int32),
        pltpu.VMEM((2, BE, DM), jnp.float32),
        pltpu.VMEM_SHARED((NPAD, DM), jnp.float32),
        [pltpu.SemaphoreType.DMA] * 4,
        [pltpu.SemaphoreType.DMA] * 2,
        pltpu.SemaphoreType.DMA,
    ],
)
def _spmm_kernel(hp_h, row_h, col_h, val_h, zero2_h, agg_out,
                 row_t, col_t, val_t, rows_t, sc_t, sh_agg, sg, ss, si):
    c = lax.axis_index("c")
    s = lax.axis_index("s")
    nwin_local = jnp.where(c == 0, WIN_C0, WIN_C1)
    chunk0 = c * (NS * WIN_C0) + s * nwin_local
    base = s * STRIPE
    for k in range(STRIPE // 128):
        pltpu.sync_copy(zero2_h, sh_agg.at[pl.ds(base + k * 128, 128)])
    plsc.subcore_barrier()

    pltpu.sync_copy(row_h.at[chunk0], row_t.at[0])
    pltpu.sync_copy(col_h.at[chunk0], col_t.at[0])
    pltpu.sync_copy(val_h.at[chunk0], val_t.at[0])
    pltpu.async_copy(row_h.at[chunk0 + 1], row_t.at[1], si)
    pltpu.async_copy(col_h.at[chunk0 + 1], col_t.at[1], si)
    pltpu.async_copy(val_h.at[chunk0 + 1], val_t.at[1], si)
    for kb in range(4):
        pltpu.async_copy(hp_h.at[col_t.at[0, kb]], rows_t.at[kb], sg[kb])

    def _drain_si(wbn):
        pltpu.make_async_copy(row_h.at[0], row_t.at[wbn], si).wait()
        pltpu.make_async_copy(col_h.at[0], col_t.at[wbn], si).wait()
        pltpu.make_async_copy(val_h.at[0], val_t.at[wbn], si).wait()

    def _win(win, carry):
        wb = win % 2
        wbn = 1 - wb

        def _grp(i, carry2):
            for kb in range(4):
                b = 4 * i + kb
                bg = win * WIN + b
                kc = kb % 2
                pltpu.make_async_copy(
                    hp_h.at[col_t.at[wb, b]], rows_t.at[kb], sg[kb]).wait()
                @pl.when(bg >= 2)
                def _():
                    pltpu.make_async_copy(
                        sc_t.at[kc], sh_agg.at[row_t.at[wb, b]], ss[kc]).wait()
                for j2 in range(BE // 16):
                    cv = val_t[wb, b, pl.ds(j2 * 16, 16)]
                    for j in range(16):
                        e = j2 * 16 + j
                        cf = cv[j]
                        for k in range(DM // 32):
                            u = rows_t[kb, e, pl.ds(k * 16, 16)]
                            lo = lax.bitcast_convert_type(
                                u << 16, jnp.float32)
                            hi = lax.bitcast_convert_type(
                                u & jnp.int32(-65536), jnp.float32)
                            sc_t[kc, e, pl.ds(k * 32, 16)] = lo * cf
                            sc_t[kc, e, pl.ds(k * 32 + 16, 16)] = hi * cf
                pltpu.async_copy(
                    sc_t.at[kc], sh_agg.at[row_t.at[wb, b]], ss[kc],
                    add=True)
                if kb == 3:
                    @pl.when((i == 0) & (win >= 1) & (win + 1 < nwin_local))
                    def _():
                        chn = chunk0 + win + 1
                        pltpu.async_copy(row_h.at[chn], row_t.at[wbn], si)
                        pltpu.async_copy(col_h.at[chn], col_t.at[wbn], si)
                        pltpu.async_copy(val_h.at[chn], val_t.at[wbn], si)
                if kb == 0:
                    @pl.when((i == 3) & (win + 1 < nwin_local))
                    def _():
                        _drain_si(wbn)
                @pl.when(i < 3)
                def _():
                    pltpu.async_copy(
                        hp_h.at[col_t.at[wb, b + 4]], rows_t.at[kb], sg[kb])
                @pl.when((i == 3) & (win + 1 < nwin_local))
                def _():
                    pltpu.async_copy(
                        hp_h.at[col_t.at[wbn, kb]], rows_t.at[kb], sg[kb])
            return carry2

        lax.fori_loop(0, WIN // 4, _grp, 0)
        return carry

    lax.fori_loop(0, nwin_local, _win, 0)
    for kc in (0, 1):
        pltpu.make_async_copy(
            sc_t.at[kc], sh_agg.at[row_t.at[0, WIN - 2 + kc]], ss[kc]).wait()
    plsc.subcore_barrier()
    for k in range(STRIPE // 128):
        off = base + k * 128
        pltpu.sync_copy(sh_agg.at[pl.ds(off, 128)],
                        agg_out.at[pl.ds(c * NPAD + off, 128)])


_RB = 1000


def _support_body(x_ref, wt_ref, b_ref, d0_ref, d1_ref, sup_ref, hp_ref, dc_ref):
    sup = jnp.dot(x_ref[...], wt_ref[...], preferred_element_type=jnp.float32)
    sup = sup + b_ref[...]
    d = lax.rsqrt(d0_ref[...] + d1_ref[...] + 1.0)
    sup_ref[...] = sup
    hp_ref[...] = sup * d
    dc_ref[...] = d


def _combine_body(a0_ref, a1_ref, hp_ref, sup_ref, dc_ref, out_ref):
    acc = (a0_ref[...] + a1_ref[...] + hp_ref[...]) * dc_ref[...]
    out_ref[...] = acc * (SMOOTH / (1.0 + SMOOTH)) + sup_ref[...] * (1.0 / (1.0 + SMOOTH))


def kernel(x, adj_indices, adj_values, W, b):
    row = adj_indices[0]
    col = adj_indices[1]
    pad = EP - E
    row_p = jnp.pad(row, (0, pad)).reshape(NW * NWIN, WIN, BE)
    col_p = jnp.pad(col, (0, pad)).reshape(NW * NWIN, WIN, BE)
    val_p = jnp.pad(adj_values, (0, pad)).reshape(NW * NWIN, WIN, BE)
    col_p4 = col_p.reshape(NW, NB, BE)
    val_p4 = val_p.reshape(NW, NB, BE)
    zero1 = jnp.zeros((NPAD,), jnp.float32)
    zero2 = jnp.zeros((128, DM), jnp.float32)

    deg_p = _deg_kernel(col_p4, val_p4, zero1)
    d0 = deg_p[0, :N, None]
    d1 = deg_p[1, :N, None]

    wt = W.T
    b2 = b.reshape(1, DM)
    grid = (N // _RB,)
    sup, hp, dc = pl.pallas_call(
        _support_body,
        grid=grid,
        in_specs=[
            pl.BlockSpec((_RB, DM), lambda i: (i, 0)),
            pl.BlockSpec((DM, DM), lambda i: (0, 0)),
            pl.BlockSpec((1, DM), lambda i: (0, 0)),
            pl.BlockSpec((_RB, 1), lambda i: (i, 0)),
            pl.BlockSpec((_RB, 1), lambda i: (i, 0)),
        ],
        out_specs=[
            pl.BlockSpec((_RB, DM), lambda i: (i, 0)),
            pl.BlockSpec((_RB, DM), lambda i: (i, 0)),
            pl.BlockSpec((_RB, 1), lambda i: (i, 0)),
        ],
        out_shape=[
            jax.ShapeDtypeStruct((N, DM), jnp.float32),
            jax.ShapeDtypeStruct((N, DM), jnp.float32),
            jax.ShapeDtypeStruct((N, 1), jnp.float32),
        ],
    )(x, wt, b2, d0, d1)

    hp_b = (hp.reshape(N, DM // 32, 2, 16).swapaxes(2, 3)
            .reshape(N, DM // 2, 2).astype(jnp.bfloat16))
    hp_i = jax.lax.bitcast_convert_type(hp_b, jnp.int32)
    agg = _spmm_kernel(hp_i, row_p, col_p, val_p, zero2)
    a0 = agg[:N]
    a1 = agg[NPAD:NPAD + N]

    out = pl.pallas_call(
        _combine_body,
        grid=grid,
        in_specs=[
            pl.BlockSpec((_RB, DM), lambda i: (i, 0)),
            pl.BlockSpec((_RB, DM), lambda i: (i, 0)),
            pl.BlockSpec((_RB, DM), lambda i: (i, 0)),
            pl.BlockSpec((_RB, DM), lambda i: (i, 0)),
            pl.BlockSpec((_RB, 1), lambda i: (i, 0)),
        ],
        out_specs=pl.BlockSpec((_RB, DM), lambda i: (i, 0)),
        out_shape=jax.ShapeDtypeStruct((N, DM), jnp.float32),
    )(a0, a1, hp, sup, dc)
    return out


# speedup vs baseline: 1.1433x; 1.0110x over previous
import functools

import jax
import jax.numpy as jnp
from jax import lax
from jax.experimental import pallas as pl
from jax.experimental.pallas import tpu as pltpu
from jax.experimental.pallas import tpu_sc as plsc

N = 10000
E = 320000
DM = 128
SMOOTH = 0.5

NC = 2
NS = 16
NW = NC * NS
BE = 64
NB = 160
WIN = 16
NWIN = NB // WIN
WIN_C0 = 13
WIN_C1 = 7
EPW = NB * BE
EP = NW * EPW
NPAD = 10240
STRIPE = NPAD // NS

_mesh = plsc.VectorSubcoreMesh(core_axis_name="c", subcore_axis_name="s")


@functools.partial(
    pl.kernel,
    mesh=_mesh,
    out_type=jax.ShapeDtypeStruct((NC, NPAD), jnp.float32),
    scratch_types=[
        pltpu.VMEM((NB, BE), jnp.int32),
        pltpu.VMEM((NB, BE), jnp.float32),
        pltpu.VMEM_SHARED((NPAD,), jnp.float32),
    ],
)
def _deg_kernel(col_h, val_h, zero1_h, deg_out, col_t, val_t, sh_deg):
    c = lax.axis_index("c")
    s = lax.axis_index("s")
    w = c * NS + s
    pltpu.sync_copy(col_h.at[w], col_t)
    pltpu.sync_copy(val_h.at[w], val_t)
    pltpu.sync_copy(zero1_h.at[pl.ds(s * STRIPE, STRIPE)],
                    sh_deg.at[pl.ds(s * STRIPE, STRIPE)])
    plsc.subcore_barrier()

    def _scat(b, carry):
        pltpu.sync_copy(val_t.at[b], sh_deg.at[col_t.at[b]], add=True)
        return carry

    lax.fori_loop(0, NB, _scat, 0)
    plsc.subcore_barrier()
    pltpu.sync_copy(sh_deg.at[pl.ds(s * STRIPE, STRIPE)],
                    deg_out.at[c, pl.ds(s * STRIPE, STRIPE)])


@functools.partial(
    pl.kernel,
    mesh=_mesh,
    out_type=jax.ShapeDtypeStruct((NC * NPAD, DM), jnp.float32),
    compiler_params=pltpu.CompilerParams(
        needs_layout_passes=False, use_tc_tiling_on_sc=False),
    scratch_types=[
        pltpu.VMEM((2, WIN, BE), jnp.int32),
        pltpu.VMEM((2, WIN, BE), jnp.int32),
        pltpu.VMEM((2, WIN, BE), jnp.float32),
        pltpu.VMEM((4, BE, DM // 2), jnp.int32),
        pltpu.VMEM((2, BE, DM), jnp.float32),
        pltpu.VMEM_SHARED((NPAD, DM), jnp.float32),
        [pltpu.SemaphoreType.DMA] * 4,
        [pltpu.SemaphoreType.DMA] * 2,
        pltpu.SemaphoreType.DMA,
    ],
)
def _spmm_kernel(hp_h, row_h, col_h, val_h, zero2_h, agg_out,
                 row_t, col_t, val_t, rows_t, sc_t, sh_agg, sg, ss, si):
    c = lax.axis_index("c")
    s = lax.axis_index("s")
    nwin_local = jnp.where(c == 0, WIN_C0, WIN_C1)
    chunk0 = c * (NS * WIN_C0) + s * nwin_local
    base = s * STRIPE
    for k in range(STRIPE // 128):
        pltpu.sync_copy(zero2_h, sh_agg.at[pl.ds(base + k * 128, 128)])
    plsc.subcore_barrier()

    pltpu.sync_copy(row_h.at[chunk0], row_t.at[0])
    pltpu.sync_copy(col_h.at[chunk0], col_t.at[0])
    pltpu.sync_copy(val_h.at[chunk0], val_t.at[0])
    pltpu.async_copy(row_h.at[chunk0 + 1], row_t.at[1], si)
    pltpu.async_copy(col_h.at[chunk0 + 1], col_t.at[1], si)
    pltpu.async_copy(val_h.at[chunk0 + 1], val_t.at[1], si)
    for kb in range(4):
        pltpu.async_copy(hp_h.at[col_t.at[0, kb]], rows_t.at[kb], sg[kb])

    def _drain_si(wbn):
        pltpu.make_async_copy(row_h.at[0], row_t.at[wbn], si).wait()
        pltpu.make_async_copy(col_h.at[0], col_t.at[wbn], si).wait()
        pltpu.make_async_copy(val_h.at[0], val_t.at[wbn], si).wait()

    def _win(win, carry):
        wb = win % 2
        wbn = 1 - wb

        def _grp(i, carry2):
            for kb in range(4):
                b = 4 * i + kb
                bg = win * WIN + b
                kc = kb % 2
                pltpu.make_async_copy(
                    hp_h.at[col_t.at[wb, b]], rows_t.at[kb], sg[kb]).wait()
                @pl.when(bg >= 2)
                def _():
                    pltpu.make_async_copy(
                        sc_t.at[kc], sh_agg.at[row_t.at[wb, b]], ss[kc]).wait()
                for j2 in range(BE // 16):
                    cv = val_t[wb, b, pl.ds(j2 * 16, 16)]
                    for j in range(16):
                        e = j2 * 16 + j
                        cf = cv[j]
                        for k in range(DM // 32):
                            u = rows_t[kb, e, pl.ds(k * 16, 16)]
                            lo = lax.bitcast_convert_type(
                                u << 16, jnp.float32)
                            hi = lax.bitcast_convert_type(
                                u & jnp.int32(-65536), jnp.float32)
                            sc_t[kc, e, pl.ds(k * 32, 16)] = lo * cf
                            sc_t[kc, e, pl.ds(k * 32 + 16, 16)] = hi * cf
                pltpu.async_copy(
                    sc_t.at[kc], sh_agg.at[row_t.at[wb, b]], ss[kc],
                    add=True)
                if kb == 3:
                    @pl.when((i == 0) & (win >= 1) & (win + 1 < nwin_local))
                    def _():
                        chn = chunk0 + win + 1
                        pltpu.async_copy(row_h.at[chn], row_t.at[wbn], si)
                        pltpu.async_copy(col_h.at[chn], col_t.at[wbn], si)
                        pltpu.async_copy(val_h.at[chn], val_t.at[wbn], si)
                if kb == 0:
                    @pl.when((i == 3) & (win + 1 < nwin_local))
                    def _():
                        _drain_si(wbn)
                @pl.when(i < 3)
                def _():
                    pltpu.async_copy(
                        hp_h.at[col_t.at[wb, b + 4]], rows_t.at[kb], sg[kb])
                @pl.when((i == 3) & (win + 1 < nwin_local))
                def _():
                    pltpu.async_copy(
                        hp_h.at[col_t.at[wbn, kb]], rows_t.at[kb], sg[kb])
            return carry2

        lax.fori_loop(0, WIN // 4, _grp, 0)
        return carry

    lax.fori_loop(0, nwin_local, _win, 0)
    for kc in (0, 1):
        pltpu.make_async_copy(
            sc_t.at[kc], sh_agg.at[row_t.at[0, WIN - 2 + kc]], ss[kc]).wait()
    plsc.subcore_barrier()
    for k in range(STRIPE // 128):
        off = base + k * 128
        pltpu.sync_copy(sh_agg.at[pl.ds(off, 128)],
                        agg_out.at[pl.ds(c * NPAD + off, 128)])


_RB = 1000


def _support_body(x_ref, wt_ref, b_ref, d0_ref, d1_ref, sup_ref, hp_ref, dc_ref):
    sup = jnp.dot(x_ref[...], wt_ref[...], preferred_element_type=jnp.float32)
    sup = sup + b_ref[...]
    d = lax.rsqrt(d0_ref[...] + d1_ref[...] + 1.0)
    sup_ref[...] = sup
    hp_ref[...] = sup * d
    dc_ref[...] = d


def _combine_body(a0_ref, a1_ref, hp_ref, sup_ref, dc_ref, out_ref):
    acc = (a0_ref[...] + a1_ref[...] + hp_ref[...]) * dc_ref[...]
    out_ref[...] = acc * (SMOOTH / (1.0 + SMOOTH)) + sup_ref[...] * (1.0 / (1.0 + SMOOTH))


def kernel(x, adj_indices, adj_values, W, b):
    row = adj_indices[0]
    col = adj_indices[1]
    pad = EP - E
    row_p = jnp.pad(row, (0, pad)).reshape(NW * NWIN, WIN, BE)
    col_p = jnp.pad(col, (0, pad)).reshape(NW * NWIN, WIN, BE)
    val_p = jnp.pad(adj_values, (0, pad)).reshape(NW * NWIN, WIN, BE)
    col_p4 = col_p.reshape(NW, NB, BE)
    val_p4 = val_p.reshape(NW, NB, BE)
    zero1 = jnp.zeros((NPAD,), jnp.float32)
    zero2 = jnp.zeros((128, DM), jnp.float32)

    deg_p = _deg_kernel(col_p4, val_p4, zero1)
    d0 = deg_p[0, :N, None]
    d1 = deg_p[1, :N, None]

    wt = W.T
    b2 = b.reshape(1, DM)
    grid = (N // _RB,)
    sup, hp, dc = pl.pallas_call(
        _support_body,
        grid=grid,
        in_specs=[
            pl.BlockSpec((_RB, DM), lambda i: (i, 0)),
            pl.BlockSpec((DM, DM), lambda i: (0, 0)),
            pl.BlockSpec((1, DM), lambda i: (0, 0)),
            pl.BlockSpec((_RB, 1), lambda i: (i, 0)),
            pl.BlockSpec((_RB, 1), lambda i: (i, 0)),
        ],
        out_specs=[
            pl.BlockSpec((_RB, DM), lambda i: (i, 0)),
            pl.BlockSpec((_RB, DM), lambda i: (i, 0)),
            pl.BlockSpec((_RB, 1), lambda i: (i, 0)),
        ],
        out_shape=[
            jax.ShapeDtypeStruct((N, DM), jnp.float32),
            jax.ShapeDtypeStruct((N, DM), jnp.float32),
            jax.ShapeDtypeStruct((N, 1), jnp.float32),
        ],
    )(x, wt, b2, d0, d1)

    hp_b = (hp.reshape(N, DM // 32, 2, 16).swapaxes(2, 3)
            .reshape(N, DM // 2, 2).astype(jnp.bfloat16))
    hp_i = jax.lax.bitcast_convert_type(hp_b, jnp.int32)
    agg = _spmm_kernel(hp_i, row_p, col_p, val_p, zero2)
    a0 = agg[:N]
    a1 = agg[NPAD:NPAD + N]

    out = pl.pallas_call(
        _combine_body,
        grid=grid,
        in_specs=[
            pl.BlockSpec((_RB, DM), lambda i: (i, 0)),
            pl.BlockSpec((_RB, DM), lambda i: (i, 0)),
            pl.BlockSpec((_RB, DM), lambda i: (i, 0)),
            pl.BlockSpec((_RB, DM), lambda i: (i, 0)),
            pl.BlockSpec((_RB, 1), lambda i: (i, 0)),
        ],
        out_specs=pl.BlockSpec((_RB, DM), lambda i: (i, 0)),
        out_shape=jax.ShapeDtypeStruct((N, DM), jnp.float32),
    )(a0, a1, hp, sup, dc)
    return out
